# baseline (speedup 1.0000x reference)
import functools

import jax
import jax.numpy as jnp
from jax import lax
from jax.experimental import pallas as pl
from jax.experimental.pallas import tpu as pltpu
from jax.experimental.pallas import tpu_sc as plsc

_IDX_CHUNK = 64


def _make_sc_gather(batch, seq, emb):
    info = plsc.get_sparse_core_info()
    n_workers = info.num_cores * info.num_subcores
    rows_per_w = batch * seq // n_workers
    n_chunks = rows_per_w // _IDX_CHUNK
    w_per_batch = seq // rows_per_w
    mesh = plsc.VectorSubcoreMesh(core_axis_name="c", subcore_axis_name="s")

    @functools.partial(
        pl.kernel,
        mesh=mesh,
        out_type=jax.ShapeDtypeStruct((batch, seq, emb), jnp.float32),
        scratch_types=[
            pltpu.VMEM((rows_per_w,), jnp.int32),
            pltpu.VMEM((rows_per_w, emb), jnp.float32),
            pltpu.SemaphoreType.DMA,
            pltpu.SemaphoreType.DMA,
        ],
    )
    def gather_rows(table_hbm, idx_hbm, out_hbm, idx_v, rows_v, gsem, wsem):
        wid = lax.axis_index("s") * info.num_cores + lax.axis_index("c")
        b = wid // w_per_batch
        s0 = (wid % w_per_batch) * rows_per_w
        pltpu.sync_copy(idx_hbm.at[b, pl.ds(s0, rows_per_w)], idx_v)
        gathers = [
            pltpu.async_copy(
                table_hbm.at[idx_v.at[pl.ds(j * _IDX_CHUNK, _IDX_CHUNK)]],
                rows_v.at[pl.ds(j * _IDX_CHUNK, _IDX_CHUNK)],
                gsem,
            )
            for j in range(n_chunks)
        ]
        writes = []
        for j in range(n_chunks):
            gathers[j].wait()
            writes.append(
                pltpu.async_copy(
                    rows_v.at[pl.ds(j * _IDX_CHUNK, _IDX_CHUNK)],
                    out_hbm.at[b, pl.ds(s0 + j * _IDX_CHUNK, _IDX_CHUNK)],
                    wsem,
                )
            )
        for w in writes:
            w.wait()

    return gather_rows


def _tc_body(emb_ref, t_ref, wt_ref, b_ref, pos_ref, tok_ref, nw_ref, nb_ref,
             out_ref):
    x = emb_ref[0].astype(jnp.bfloat16)
    s, e = x.shape
    z = jnp.zeros((1, e), jnp.bfloat16)
    left = jnp.concatenate([x[1:], z], axis=0)
    right = jnp.concatenate([z, x[:-1]], axis=0)
    tri = jnp.concatenate([left, x, right], axis=1)
    p = jnp.dot(tri, wt_ref[...], preferred_element_type=jnp.float32)
    t = t_ref[0]
    tok0 = tok_ref[0:1, :]
    tok_emb = tok0 + t * (tok_ref[1:2, :] - tok0)
    res = p + b_ref[...] + pos_ref[...].astype(jnp.float32) + tok_emb
    out_ref[0] = res * nw_ref[...] + nb_ref[...]


def kernel(input_ids, token_type_ids, word_table, lin_w, lin_b, pos_table,
           tok_table, norm_w, norm_b):
    batch, seq = input_ids.shape
    vocab, emb = word_table.shape
    hid = lin_w.shape[0]

    emb3 = _make_sc_gather(batch, seq, emb)(word_table, input_ids)

    t_col = token_type_ids.astype(jnp.float32).reshape(batch, seq, 1)
    w_t = lin_w.T.astype(jnp.bfloat16)
    pos_bf = pos_table.astype(jnp.bfloat16)
    b_row = lin_b.reshape(1, hid)
    nw_row = norm_w.reshape(1, hid)
    nb_row = norm_b.reshape(1, hid)

    out = pl.pallas_call(
        _tc_body,
        grid=(batch,),
        in_specs=[
            pl.BlockSpec((1, seq, emb), lambda b: (b, 0, 0)),
            pl.BlockSpec((1, seq, 1), lambda b: (b, 0, 0)),
            pl.BlockSpec((3 * emb, hid), lambda b: (0, 0)),
            pl.BlockSpec((1, hid), lambda b: (0, 0)),
            pl.BlockSpec((seq, hid), lambda b: (0, 0)),
            pl.BlockSpec(tok_table.shape, lambda b: (0, 0)),
            pl.BlockSpec((1, hid), lambda b: (0, 0)),
            pl.BlockSpec((1, hid), lambda b: (0, 0)),
        ],
        out_specs=pl.BlockSpec((1, seq, hid), lambda b: (b, 0, 0)),
        out_shape=jax.ShapeDtypeStruct((batch, seq, hid), jnp.float32),
        compiler_params=pltpu.CompilerParams(
            dimension_semantics=("parallel",),
        ),
    )(emb3, t_col, w_t, b_row, pos_bf, tok_table, nw_row, nb_row)
    return out


# speedup vs baseline: 1.0058x; 1.0058x over previous
import functools

import jax
import jax.numpy as jnp
from jax import lax
from jax.experimental import pallas as pl
from jax.experimental.pallas import tpu as pltpu
from jax.experimental.pallas import tpu_sc as plsc

_IDX_CHUNK = 128


def _make_sc_gather(batch, seq, emb):
    info = plsc.get_sparse_core_info()
    n_workers = info.num_cores * info.num_subcores
    rows_per_w = batch * seq // n_workers
    n_chunks = rows_per_w // _IDX_CHUNK
    w_per_batch = seq // rows_per_w
    mesh = plsc.VectorSubcoreMesh(core_axis_name="c", subcore_axis_name="s")

    @functools.partial(
        pl.kernel,
        mesh=mesh,
        out_type=jax.ShapeDtypeStruct((batch, seq, emb), jnp.float32),
        scratch_types=[
            pltpu.VMEM((rows_per_w,), jnp.int32),
            pltpu.VMEM((rows_per_w, emb), jnp.float32),
            pltpu.SemaphoreType.DMA,
            pltpu.SemaphoreType.DMA,
        ],
    )
    def gather_rows(table_hbm, idx_hbm, out_hbm, idx_v, rows_v, gsem, wsem):
        wid = lax.axis_index("s") * info.num_cores + lax.axis_index("c")
        b = wid // w_per_batch
        s0 = (wid % w_per_batch) * rows_per_w
        pltpu.sync_copy(idx_hbm.at[b, pl.ds(s0, rows_per_w)], idx_v)
        gathers = [
            pltpu.async_copy(
                table_hbm.at[idx_v.at[pl.ds(j * _IDX_CHUNK, _IDX_CHUNK)]],
                rows_v.at[pl.ds(j * _IDX_CHUNK, _IDX_CHUNK)],
                gsem,
            )
            for j in range(n_chunks)
        ]
        writes = []
        for j in range(n_chunks):
            gathers[j].wait()
            writes.append(
                pltpu.async_copy(
                    rows_v.at[pl.ds(j * _IDX_CHUNK, _IDX_CHUNK)],
                    out_hbm.at[b, pl.ds(s0 + j * _IDX_CHUNK, _IDX_CHUNK)],
                    wsem,
                )
            )
        for w in writes:
            w.wait()

    return gather_rows


def _tc_body(emb_ref, t_ref, wt_ref, b_ref, pos_ref, tok_ref, nw_ref, nb_ref,
             out_ref):
    x = emb_ref[0].astype(jnp.bfloat16)
    s, e = x.shape
    z = jnp.zeros((1, e), jnp.bfloat16)
    left = jnp.concatenate([x[1:], z], axis=0)
    right = jnp.concatenate([z, x[:-1]], axis=0)
    tri = jnp.concatenate([left, x, right], axis=1)
    p = jnp.dot(tri, wt_ref[...], preferred_element_type=jnp.float32)
    t = t_ref[0]
    tok0 = tok_ref[0:1, :]
    tok_emb = tok0 + t * (tok_ref[1:2, :] - tok0)
    res = p + b_ref[...] + pos_ref[...].astype(jnp.float32) + tok_emb
    out_ref[0] = res * nw_ref[...] + nb_ref[...]


def kernel(input_ids, token_type_ids, word_table, lin_w, lin_b, pos_table,
           tok_table, norm_w, norm_b):
    batch, seq = input_ids.shape
    vocab, emb = word_table.shape
    hid = lin_w.shape[0]

    emb3 = _make_sc_gather(batch, seq, emb)(word_table, input_ids)

    t_col = token_type_ids.astype(jnp.float32).reshape(batch, seq, 1)
    w_t = lin_w.T.astype(jnp.bfloat16)
    pos_bf = pos_table.astype(jnp.bfloat16)
    b_row = lin_b.reshape(1, hid)
    nw_row = norm_w.reshape(1, hid)
    nb_row = norm_b.reshape(1, hid)

    out = pl.pallas_call(
        _tc_body,
        grid=(batch,),
        in_specs=[
            pl.BlockSpec((1, seq, emb), lambda b: (b, 0, 0)),
            pl.BlockSpec((1, seq, 1), lambda b: (b, 0, 0)),
            pl.BlockSpec((3 * emb, hid), lambda b: (0, 0)),
            pl.BlockSpec((1, hid), lambda b: (0, 0)),
            pl.BlockSpec((seq, hid), lambda b: (0, 0)),
            pl.BlockSpec(tok_table.shape, lambda b: (0, 0)),
            pl.BlockSpec((1, hid), lambda b: (0, 0)),
            pl.BlockSpec((1, hid), lambda b: (0, 0)),
        ],
        out_specs=pl.BlockSpec((1, seq, hid), lambda b: (b, 0, 0)),
        out_shape=jax.ShapeDtypeStruct((batch, seq, hid), jnp.float32),
        compiler_params=pltpu.CompilerParams(
            dimension_semantics=("parallel",),
        ),
    )(emb3, t_col, w_t, b_row, pos_bf, tok_table, nw_row, nb_row)
    return out
